# baseline (speedup 1.0000x reference)
import jax
import jax.numpy as jnp
from jax.experimental import pallas as pl
from jax.experimental.pallas import tpu as pltpu

HIDDEN = 1024
HEADS = 16
ATT = HIDDEN // HEADS
SCALE = ATT ** -0.5


def _mha_kernel(q_ref, k0_ref, k1_ref, v0_ref, v1_ref, wq_ref, bq_ref, wk_ref,
                wv_ref, bv_ref, wo_ref, bo_ref, out_ref, u_ref, a_ref, s_ref,
                upd_ref, wv_s, wo_s):
    b = pl.program_id(0)
    B = q_ref.shape[0]

    @pl.when(b == 0)
    def _prologue():
        qh = jax.lax.dot_general(q_ref[...], wq_ref[...],
                                 (((1,), (0,)), ((), ())),
                                 preferred_element_type=jnp.float32)
        qh = (qh + bq_ref[...]) * SCALE
        for h in range(HEADS):
            qs = qh[:, h * ATT:(h + 1) * ATT]
            ws = wk_ref[:, h * ATT:(h + 1) * ATT]
            u_ref[:, h, :] = jax.lax.dot_general(
                qs, ws, (((1,), (1,)), ((), ())),
                preferred_element_type=jnp.float32)

    WC = HIDDEN // 8

    @pl.when(b < 8)
    def _stage_wv():
        wv_s[pl.ds(b * WC, WC), :] = wv_ref[...]

    @pl.when(jnp.logical_and(b >= 8, b < 16))
    def _stage_wo():
        wo_s[pl.ds((b - 8) * WC, WC), :] = wo_ref[...]

    u = u_ref[b]
    s0 = jax.lax.dot_general(k0_ref[0], u, (((1,), (1,)), ((), ())),
                             preferred_element_type=jnp.float32)
    s1 = jax.lax.dot_general(k1_ref[0], u, (((1,), (1,)), ((), ())),
                             preferred_element_type=jnp.float32)
    m = jnp.maximum(jnp.max(s0, axis=0, keepdims=True),
                    jnp.max(s1, axis=0, keepdims=True))
    e0 = jnp.exp(s0 - m)
    e1 = jnp.exp(s1 - m)
    s_ref[b] = (jnp.sum(e0, axis=0, keepdims=True)
                + jnp.sum(e1, axis=0, keepdims=True))
    a_ref[b] = (
        jax.lax.dot_general(e0, v0_ref[0], (((0,), (0,)), ((), ())),
                            preferred_element_type=jnp.float32)
        + jax.lax.dot_general(e1, v1_ref[0], (((0,), (0,)), ((), ())),
                              preferred_element_type=jnp.float32))

    @pl.when(b == B - 1)
    def _epilogue():
        r = 1.0 / s_ref[...]
        for h in range(HEADS):
            ah = a_ref[:, h, :] * r[:, 0, h:h + 1]
            ws = wv_s[:, h * ATT:(h + 1) * ATT]
            upd_ref[:, h * ATT:(h + 1) * ATT] = jax.lax.dot_general(
                ah, ws, (((1,), (0,)), ((), ())),
                preferred_element_type=jnp.float32) + bv_ref[:, h * ATT:(h + 1) * ATT]
        out_ref[...] = jax.lax.dot_general(
            upd_ref[...], wo_s[...], (((1,), (0,)), ((), ())),
            preferred_element_type=jnp.float32) + bo_ref[...]


def kernel(q, k, v, Wq, bq, Wk, bk, Wv, bv, Wo, bo):
    del bk
    B = q.shape[0]
    L_K = k.shape[1]
    L2 = L_K // 2
    full = lambda b: (0, 0)
    return pl.pallas_call(
        _mha_kernel,
        grid=(B,),
        in_specs=[
            pl.BlockSpec((B, HIDDEN), full),
            pl.BlockSpec((1, L2, HIDDEN), lambda b: (b, 0, 0)),
            pl.BlockSpec((1, L2, HIDDEN), lambda b: (b, 1, 0)),
            pl.BlockSpec((1, L2, HIDDEN), lambda b: (b, 0, 0)),
            pl.BlockSpec((1, L2, HIDDEN), lambda b: (b, 1, 0)),
            pl.BlockSpec((HIDDEN, HIDDEN), full),
            pl.BlockSpec((1, HIDDEN), full),
            pl.BlockSpec((HIDDEN, HIDDEN), full),
            pl.BlockSpec((HIDDEN // 8, HIDDEN),
                         lambda b: (jnp.minimum(b, 7), 0)),
            pl.BlockSpec((1, HIDDEN), full),
            pl.BlockSpec((HIDDEN // 8, HIDDEN),
                         lambda b: (jnp.clip(b - 8, 0, 7), 0)),
            pl.BlockSpec((1, HIDDEN), full),
        ],
        out_specs=pl.BlockSpec((B, HIDDEN), full),
        out_shape=jax.ShapeDtypeStruct((B, HIDDEN), jnp.float32),
        scratch_shapes=[
            pltpu.VMEM((B, HEADS, HIDDEN), jnp.float32),
            pltpu.VMEM((B, HEADS, HIDDEN), jnp.float32),
            pltpu.VMEM((B, 1, HEADS), jnp.float32),
            pltpu.VMEM((B, HIDDEN), jnp.float32),
            pltpu.VMEM((HIDDEN, HIDDEN), jnp.float32),
            pltpu.VMEM((HIDDEN, HIDDEN), jnp.float32),
        ],
    )(q, k, k, v, v, Wq, bq.reshape(1, HIDDEN), Wk, Wv,
      bv.reshape(1, HIDDEN), Wo, bo.reshape(1, HIDDEN))


# speedup vs baseline: 1.0059x; 1.0059x over previous
import jax
import jax.numpy as jnp
from jax.experimental import pallas as pl
from jax.experimental.pallas import tpu as pltpu

HIDDEN = 1024
HEADS = 16
ATT = HIDDEN // HEADS
SCALE = ATT ** -0.5


def _mha_kernel(q_ref, k0_ref, k1_ref, v0_ref, v1_ref, wq_ref, bq_ref, wk_ref,
                wv_ref, bv_ref, wo_ref, bo_ref, out_ref, u_ref, a_ref, s_ref,
                upd_ref):
    b = pl.program_id(0)
    B = q_ref.shape[0]

    @pl.when(b == 0)
    def _prologue():
        qh = jax.lax.dot_general(q_ref[...], wq_ref[...],
                                 (((1,), (0,)), ((), ())),
                                 preferred_element_type=jnp.float32)
        qh = (qh + bq_ref[...]) * SCALE
        for h in range(HEADS):
            qs = qh[:, h * ATT:(h + 1) * ATT]
            ws = wk_ref[:, h * ATT:(h + 1) * ATT]
            u_ref[:, h, :] = jax.lax.dot_general(
                qs, ws, (((1,), (1,)), ((), ())),
                preferred_element_type=jnp.float32)

    u = u_ref[b]
    s0 = jax.lax.dot_general(k0_ref[0], u, (((1,), (1,)), ((), ())),
                             preferred_element_type=jnp.float32)
    s1 = jax.lax.dot_general(k1_ref[0], u, (((1,), (1,)), ((), ())),
                             preferred_element_type=jnp.float32)
    m = jnp.maximum(jnp.max(s0, axis=0, keepdims=True),
                    jnp.max(s1, axis=0, keepdims=True))
    e0 = jnp.exp(s0 - m)
    e1 = jnp.exp(s1 - m)
    s_ref[b] = (jnp.sum(e0, axis=0, keepdims=True)
                + jnp.sum(e1, axis=0, keepdims=True))
    a_ref[b] = (
        jax.lax.dot_general(e0, v0_ref[0], (((0,), (0,)), ((), ())),
                            preferred_element_type=jnp.float32)
        + jax.lax.dot_general(e1, v1_ref[0], (((0,), (0,)), ((), ())),
                              preferred_element_type=jnp.float32))

    @pl.when(b == B - 1)
    def _epilogue():
        r = 1.0 / s_ref[...]
        for h in range(HEADS):
            ah = a_ref[:, h, :] * r[:, 0, h:h + 1]
            ws = wv_ref[:, h * ATT:(h + 1) * ATT]
            upd_ref[:, h * ATT:(h + 1) * ATT] = jax.lax.dot_general(
                ah, ws, (((1,), (0,)), ((), ())),
                preferred_element_type=jnp.float32) + bv_ref[:, h * ATT:(h + 1) * ATT]
        out_ref[...] = jax.lax.dot_general(
            upd_ref[...], wo_ref[...], (((1,), (0,)), ((), ())),
            preferred_element_type=jnp.float32) + bo_ref[...]


def kernel(q, k, v, Wq, bq, Wk, bk, Wv, bv, Wo, bo):
    del bk
    B = q.shape[0]
    L_K = k.shape[1]
    L2 = L_K // 2
    full = lambda b: (0, 0)
    return pl.pallas_call(
        _mha_kernel,
        grid=(B,),
        in_specs=[
            pl.BlockSpec((B, HIDDEN), full),
            pl.BlockSpec((1, L2, HIDDEN), lambda b: (b, 0, 0)),
            pl.BlockSpec((1, L2, HIDDEN), lambda b: (b, 1, 0)),
            pl.BlockSpec((1, L2, HIDDEN), lambda b: (b, 0, 0)),
            pl.BlockSpec((1, L2, HIDDEN), lambda b: (b, 1, 0)),
            pl.BlockSpec((HIDDEN, HIDDEN), full),
            pl.BlockSpec((1, HIDDEN), full),
            pl.BlockSpec((HIDDEN, HIDDEN), full),
            pl.BlockSpec((HIDDEN, HIDDEN), full),
            pl.BlockSpec((1, HIDDEN), full),
            pl.BlockSpec((HIDDEN, HIDDEN), full),
            pl.BlockSpec((1, HIDDEN), full),
        ],
        out_specs=pl.BlockSpec((B, HIDDEN), full),
        out_shape=jax.ShapeDtypeStruct((B, HIDDEN), jnp.float32),
        scratch_shapes=[
            pltpu.VMEM((B, HEADS, HIDDEN), jnp.float32),
            pltpu.VMEM((B, HEADS, HIDDEN), jnp.float32),
            pltpu.VMEM((B, 1, HEADS), jnp.float32),
            pltpu.VMEM((B, HIDDEN), jnp.float32),
        ],
    )(q, k, k, v, v, Wq, bq.reshape(1, HIDDEN), Wk, Wv,
      bv.reshape(1, HIDDEN), Wo, bo.reshape(1, HIDDEN))


# speedup vs baseline: 1.0095x; 1.0036x over previous
import jax
import jax.numpy as jnp
from jax.experimental import pallas as pl
from jax.experimental.pallas import tpu as pltpu

HIDDEN = 1024
HEADS = 16
ATT = HIDDEN // HEADS
SCALE = ATT ** -0.5


def _mha_kernel(q_ref, k0_ref, k1_ref, v0_ref, v1_ref, wq_ref, bq_ref, wk_ref,
                wv_ref, bv_ref, wo_ref, bo_ref, out_ref, u_ref, a_ref, s_ref,
                upd_ref):
    b = pl.program_id(0)
    B = q_ref.shape[0]

    @pl.when(b == 0)
    def _prologue():
        qh = jax.lax.dot_general(q_ref[...], wq_ref[...],
                                 (((1,), (0,)), ((), ())),
                                 preferred_element_type=jnp.float32)
        qh = (qh + bq_ref[...]) * SCALE
        lo = (jax.lax.broadcasted_iota(jnp.int32, (1, 2 * ATT), 1)
              < ATT).astype(jnp.float32)
        for p in range(HEADS // 2):
            qs = qh[:, p * 2 * ATT:(p + 1) * 2 * ATT]
            ws = wk_ref[:, p * 2 * ATT:(p + 1) * 2 * ATT]
            u_ref[:, 2 * p, :] = jax.lax.dot_general(
                qs * lo, ws, (((1,), (1,)), ((), ())),
                preferred_element_type=jnp.float32)
            u_ref[:, 2 * p + 1, :] = jax.lax.dot_general(
                qs * (1.0 - lo), ws, (((1,), (1,)), ((), ())),
                preferred_element_type=jnp.float32)

    u = u_ref[b]
    s0 = jax.lax.dot_general(k0_ref[0], u, (((1,), (1,)), ((), ())),
                             preferred_element_type=jnp.float32)
    s1 = jax.lax.dot_general(k1_ref[0], u, (((1,), (1,)), ((), ())),
                             preferred_element_type=jnp.float32)
    m = jnp.maximum(jnp.max(s0, axis=0, keepdims=True),
                    jnp.max(s1, axis=0, keepdims=True))
    e0 = jnp.exp(s0 - m)
    e1 = jnp.exp(s1 - m)
    s_ref[b] = (jnp.sum(e0, axis=0, keepdims=True)
                + jnp.sum(e1, axis=0, keepdims=True))
    a_ref[b] = (
        jax.lax.dot_general(e0, v0_ref[0], (((0,), (0,)), ((), ())),
                            preferred_element_type=jnp.float32)
        + jax.lax.dot_general(e1, v1_ref[0], (((0,), (0,)), ((), ())),
                              preferred_element_type=jnp.float32))

    @pl.when(b == B - 1)
    def _epilogue():
        r = 1.0 / s_ref[...]
        lo = (jax.lax.broadcasted_iota(jnp.int32, (1, 2 * ATT), 1)
              < ATT).astype(jnp.float32)
        for p in range(HEADS // 2):
            a0 = a_ref[:, 2 * p, :] * r[:, 0, 2 * p:2 * p + 1]
            a1 = a_ref[:, 2 * p + 1, :] * r[:, 0, 2 * p + 1:2 * p + 2]
            ws = wv_ref[:, p * 2 * ATT:(p + 1) * 2 * ATT]
            t0 = jax.lax.dot_general(a0, ws, (((1,), (0,)), ((), ())),
                                     preferred_element_type=jnp.float32)
            t1 = jax.lax.dot_general(a1, ws, (((1,), (0,)), ((), ())),
                                     preferred_element_type=jnp.float32)
            upd_ref[:, p * 2 * ATT:(p + 1) * 2 * ATT] = (
                t0 * lo + t1 * (1.0 - lo)
                + bv_ref[:, p * 2 * ATT:(p + 1) * 2 * ATT])
        out_ref[...] = jax.lax.dot_general(
            upd_ref[...], wo_ref[...], (((1,), (0,)), ((), ())),
            preferred_element_type=jnp.float32) + bo_ref[...]


def kernel(q, k, v, Wq, bq, Wk, bk, Wv, bv, Wo, bo):
    del bk
    B = q.shape[0]
    L_K = k.shape[1]
    L2 = L_K // 2
    full = lambda b: (0, 0)
    return pl.pallas_call(
        _mha_kernel,
        grid=(B,),
        in_specs=[
            pl.BlockSpec((B, HIDDEN), full),
            pl.BlockSpec((1, L2, HIDDEN), lambda b: (b, 0, 0)),
            pl.BlockSpec((1, L2, HIDDEN), lambda b: (b, 1, 0)),
            pl.BlockSpec((1, L2, HIDDEN), lambda b: (b, 0, 0)),
            pl.BlockSpec((1, L2, HIDDEN), lambda b: (b, 1, 0)),
            pl.BlockSpec((HIDDEN, HIDDEN), full),
            pl.BlockSpec((1, HIDDEN), full),
            pl.BlockSpec((HIDDEN, HIDDEN), full),
            pl.BlockSpec((HIDDEN, HIDDEN), full),
            pl.BlockSpec((1, HIDDEN), full),
            pl.BlockSpec((HIDDEN, HIDDEN), full),
            pl.BlockSpec((1, HIDDEN), full),
        ],
        out_specs=pl.BlockSpec((B, HIDDEN), full),
        out_shape=jax.ShapeDtypeStruct((B, HIDDEN), jnp.float32),
        scratch_shapes=[
            pltpu.VMEM((B, HEADS, HIDDEN), jnp.float32),
            pltpu.VMEM((B, HEADS, HIDDEN), jnp.float32),
            pltpu.VMEM((B, 1, HEADS), jnp.float32),
            pltpu.VMEM((B, HIDDEN), jnp.float32),
        ],
    )(q, k, k, v, v, Wq, bq.reshape(1, HIDDEN), Wk, Wv,
      bv.reshape(1, HIDDEN), Wo, bo.reshape(1, HIDDEN))


# speedup vs baseline: 1.0116x; 1.0020x over previous
import jax
import jax.numpy as jnp
from jax.experimental import pallas as pl
from jax.experimental.pallas import tpu as pltpu

HIDDEN = 1024
HEADS = 16
ATT = HIDDEN // HEADS
SCALE = ATT ** -0.5


def _mha_kernel(q_ref, k0_ref, k1_ref, k2_ref, k3_ref, v0_ref, v1_ref, v2_ref,
                v3_ref, wq_ref, bq_ref, wk_ref, wv_ref, bv_ref, wo_ref, bo_ref,
                out_ref, u_ref, a_ref, s_ref, upd_ref):
    b = pl.program_id(0)
    B = q_ref.shape[0]

    @pl.when(b == 0)
    def _prologue():
        qh = jax.lax.dot_general(q_ref[...], wq_ref[...],
                                 (((1,), (0,)), ((), ())),
                                 preferred_element_type=jnp.float32)
        qh = (qh + bq_ref[...]) * SCALE
        lo = (jax.lax.broadcasted_iota(jnp.int32, (1, 2 * ATT), 1)
              < ATT).astype(jnp.float32)
        for p in range(HEADS // 2):
            qs = qh[:, p * 2 * ATT:(p + 1) * 2 * ATT]
            ws = wk_ref[:, p * 2 * ATT:(p + 1) * 2 * ATT]
            u_ref[:, 2 * p, :] = jax.lax.dot_general(
                qs * lo, ws, (((1,), (1,)), ((), ())),
                preferred_element_type=jnp.float32)
            u_ref[:, 2 * p + 1, :] = jax.lax.dot_general(
                qs * (1.0 - lo), ws, (((1,), (1,)), ((), ())),
                preferred_element_type=jnp.float32)

    u = u_ref[b]
    kq = (k0_ref, k1_ref, k2_ref, k3_ref)
    vq = (v0_ref, v1_ref, v2_ref, v3_ref)
    sc = [jax.lax.dot_general(kr[0], u, (((1,), (1,)), ((), ())),
                              preferred_element_type=jnp.float32)
          for kr in kq]
    m = jnp.max(jnp.stack([jnp.max(s, axis=0, keepdims=True) for s in sc]),
                axis=0)
    es = [jnp.exp(s - m) for s in sc]
    s_ref[b] = sum(jnp.sum(e, axis=0, keepdims=True) for e in es)
    a_ref[b] = sum(
        jax.lax.dot_general(e, vr[0], (((0,), (0,)), ((), ())),
                            preferred_element_type=jnp.float32)
        for e, vr in zip(es, vq))

    @pl.when(b == B - 1)
    def _epilogue():
        r = 1.0 / s_ref[...]
        lo = (jax.lax.broadcasted_iota(jnp.int32, (1, 2 * ATT), 1)
              < ATT).astype(jnp.float32)
        for p in range(HEADS // 2):
            a0 = a_ref[:, 2 * p, :] * r[:, 0, 2 * p:2 * p + 1]
            a1 = a_ref[:, 2 * p + 1, :] * r[:, 0, 2 * p + 1:2 * p + 2]
            ws = wv_ref[:, p * 2 * ATT:(p + 1) * 2 * ATT]
            t0 = jax.lax.dot_general(a0, ws, (((1,), (0,)), ((), ())),
                                     preferred_element_type=jnp.float32)
            t1 = jax.lax.dot_general(a1, ws, (((1,), (0,)), ((), ())),
                                     preferred_element_type=jnp.float32)
            upd_ref[:, p * 2 * ATT:(p + 1) * 2 * ATT] = (
                t0 * lo + t1 * (1.0 - lo)
                + bv_ref[:, p * 2 * ATT:(p + 1) * 2 * ATT])
        out_ref[...] = jax.lax.dot_general(
            upd_ref[...], wo_ref[...], (((1,), (0,)), ((), ())),
            preferred_element_type=jnp.float32) + bo_ref[...]


def kernel(q, k, v, Wq, bq, Wk, bk, Wv, bv, Wo, bo):
    del bk
    B = q.shape[0]
    L_K = k.shape[1]
    L4 = L_K // 4
    full = lambda b: (0, 0)
    return pl.pallas_call(
        _mha_kernel,
        grid=(B,),
        in_specs=[
            pl.BlockSpec((B, HIDDEN), full),
            pl.BlockSpec((1, L4, HIDDEN), lambda b: (b, 0, 0)),
            pl.BlockSpec((1, L4, HIDDEN), lambda b: (b, 1, 0)),
            pl.BlockSpec((1, L4, HIDDEN), lambda b: (b, 2, 0)),
            pl.BlockSpec((1, L4, HIDDEN), lambda b: (b, 3, 0)),
            pl.BlockSpec((1, L4, HIDDEN), lambda b: (b, 0, 0)),
            pl.BlockSpec((1, L4, HIDDEN), lambda b: (b, 1, 0)),
            pl.BlockSpec((1, L4, HIDDEN), lambda b: (b, 2, 0)),
            pl.BlockSpec((1, L4, HIDDEN), lambda b: (b, 3, 0)),
            pl.BlockSpec((HIDDEN, HIDDEN), full),
            pl.BlockSpec((1, HIDDEN), full),
            pl.BlockSpec((HIDDEN, HIDDEN), full),
            pl.BlockSpec((HIDDEN, HIDDEN), full),
            pl.BlockSpec((1, HIDDEN), full),
            pl.BlockSpec((HIDDEN, HIDDEN), full),
            pl.BlockSpec((1, HIDDEN), full),
        ],
        out_specs=pl.BlockSpec((B, HIDDEN), full),
        out_shape=jax.ShapeDtypeStruct((B, HIDDEN), jnp.float32),
        scratch_shapes=[
            pltpu.VMEM((B, HEADS, HIDDEN), jnp.float32),
            pltpu.VMEM((B, HEADS, HIDDEN), jnp.float32),
            pltpu.VMEM((B, 1, HEADS), jnp.float32),
            pltpu.VMEM((B, HIDDEN), jnp.float32),
        ],
    )(q, k, k, k, k, v, v, v, v, Wq, bq.reshape(1, HIDDEN), Wk, Wv,
      bv.reshape(1, HIDDEN), Wo, bo.reshape(1, HIDDEN))
